# baseline (speedup 1.0000x reference)
import jax
import jax.numpy as jnp
from jax import lax
from jax.experimental import pallas as pl
from jax.experimental.pallas import tpu as pltpu
from jax.experimental.pallas import tpu_sc as plsc

N = 10000
E = 160000
D = 256
H = 128
L = 3
NP = 10240
BLK = 512
NB = NP // BLK
SW = 128
NT = 16
NPT = NP // NT
EPT = E // NT
CH = 40
NCH = EPT // CH
EPS = 1e-5
F32 = jnp.float32

def _mesh():
    return plsc.VectorSubcoreMesh(core_axis_name="c", subcore_axis_name="s",
                                  num_cores=2, num_subcores=NT)


def _sc_init_body(temb_h, temb_d, temb_s, apad, upd, ups,
                  h0, t0d, t0s,
                  aidx, aadj, rows, urows, sem):
    c = lax.axis_index("c")
    s = lax.axis_index("s")
    nb = s * NPT

    def _add_rows(r, _):
        for g in range(D // 16):
            sl = pl.ds(g * 16, 16)
            rows[r, sl] = rows[r, sl] + urows[r, sl]
        return 0

    def chunk(j, _):
        off = nb + j * 128
        pltpu.sync_copy(apad.at[pl.ds(off, 128)], aidx)

        @pl.when(c == 0)
        def _():
            pltpu.async_copy(temb_h.at[aidx], rows, sem).wait()
            pltpu.sync_copy(rows, h0.at[pl.ds(off, 128)])

        for g in range(8):
            sl = pl.ds(g * 16, 16)
            aadj[sl] = aidx[sl] + c * 120

        pltpu.async_copy(temb_d.at[aadj], rows, sem).wait()
        pltpu.sync_copy(upd.at[pl.ds(c * NP + off, 128)], urows)
        lax.fori_loop(0, 128, _add_rows, 0)
        pltpu.sync_copy(rows, t0d.at[pl.ds(c * NP + off, 128)])

        pltpu.async_copy(temb_s.at[aadj], rows, sem).wait()
        pltpu.sync_copy(ups.at[pl.ds(c * NP + off, 128)], urows)
        lax.fori_loop(0, 128, _add_rows, 0)
        pltpu.sync_copy(rows, t0s.at[pl.ds(c * NP + off, 128)])
        return 0

    lax.fori_loop(0, NPT // 128, chunk, 0)


def _sc_edge_body(tdst, tsrc, dsti, srci, zer,
                  s_out, m2_out,
                  stab, idr, ids, ida, isa, gd, gs, mb, acc, sem):
    c = lax.axis_index("c")
    s = lax.axis_index("s")

    pltpu.sync_copy(zer.at[pl.ds(s * NPT, NPT)], stab.at[pl.ds(s * NPT, NPT)])
    zv = jnp.zeros((16,), F32)
    for g in range(8):
        acc[pl.ds(g * 16, 16)] = zv
    plsc.subcore_barrier()

    base = s * EPT
    coff = c * NP
    c1, c2, c3, c4, c5 = (1.0 / 3, 1.0 / 5, 1.0 / 7, 1.0 / 9, 1.0 / 11)

    def edge(e, _):
        for g in range(8):
            fo = pl.ds(g * 16, 16)
            so = pl.ds(H + g * 16, 16)
            f = gd[e, fo] + gs[e, fo]
            sv = gd[e, so] + gs[e, so]
            sig = 1.0 / (1.0 + jnp.exp(-f))
            t = jnp.exp(-jnp.abs(sv))
            z = t / (2.0 + t)
            z2 = z * z
            l1p = 2.0 * z * (1.0 + z2 * (c1 + z2 * (c2 + z2 * (c3 + z2 * (c4 + z2 * c5)))))
            m = sig * (jnp.maximum(sv, 0.0) + l1p)
            mb[e, fo] = m
            acc[fo] = acc[fo] + m * m
        return 0

    def chunk(j, _):
        off = base + j * CH
        pltpu.sync_copy(dsti.at[pl.ds(off, CH)], idr)
        pltpu.sync_copy(srci.at[pl.ds(off, CH)], ids)
        for o in (0, 16, 24):
            sl = pl.ds(o, 16)
            ida[sl] = idr[sl] + coff
            isa[sl] = ids[sl] + coff
        pltpu.async_copy(tdst.at[ida], gd, sem).wait()
        pltpu.async_copy(tsrc.at[isa], gs, sem).wait()
        lax.fori_loop(0, CH, edge, 0)
        pltpu.sync_copy(mb, stab.at[idr], add=True)
        return 0

    lax.fori_loop(0, NCH, chunk, 0)
    plsc.subcore_barrier()

    pltpu.sync_copy(stab.at[pl.ds(s * NPT, NPT)],
                    s_out.at[pl.ds(coff + s * NPT, NPT)])
    pltpu.sync_copy(acc, m2_out.at[pl.ds((c * NT + s) * H, H)])


def _sc_count_body(dsti, zer, cnt_out, ctab, idr, ones, sem):
    c = lax.axis_index("c")
    s = lax.axis_index("s")

    @pl.when(c == 0)
    def _():
        pltpu.sync_copy(zer.at[pl.ds(s * NPT, NPT)],
                        ctab.at[pl.ds(s * NPT, NPT)])
        onev = jnp.where(lax.iota(jnp.int32, 16) == 0, 1.0, 0.0).astype(F32)
        zv = jnp.zeros((16,), F32)

        def _initrow(e, _):
            ones[e, pl.ds(0, 16)] = onev
            for g in range(1, 8):
                ones[e, pl.ds(g * 16, 16)] = zv
            return 0

        lax.fori_loop(0, CH, _initrow, 0)
        plsc.subcore_barrier()
        base = s * EPT

        def chunk(j, _):
            pltpu.sync_copy(dsti.at[pl.ds(base + j * CH, CH)], idr)
            pltpu.sync_copy(ones, ctab.at[idr], add=True)
            return 0

        lax.fori_loop(0, NCH, chunk, 0)
        plsc.subcore_barrier()
        pltpu.sync_copy(ctab.at[pl.ds(s * NPT, NPT)],
                        cnt_out.at[pl.ds(s * NPT, NPT)])


def _sc_count(dsti, zer):
    return pl.kernel(
        _sc_count_body,
        mesh=_mesh(),
        out_type=[jax.ShapeDtypeStruct((NP, SW), F32)],
        scratch_types=[
            pltpu.VMEM_SHARED((NP, SW), F32),
            pltpu.VMEM((CH,), jnp.int32),
            pltpu.VMEM((CH, SW), F32),
            pltpu.SemaphoreType.DMA,
        ],
    )(dsti, zer)


def _sc_init(temb_h, temb_d, temb_s, apad, upd, ups):
    return pl.kernel(
        _sc_init_body,
        mesh=_mesh(),
        out_type=[
            jax.ShapeDtypeStruct((NP, D), F32),
            jax.ShapeDtypeStruct((2 * NP, D), F32),
            jax.ShapeDtypeStruct((2 * NP, D), F32),
        ],
        scratch_types=[
            pltpu.VMEM((128,), jnp.int32),
            pltpu.VMEM((128,), jnp.int32),
            pltpu.VMEM((128, D), F32),
            pltpu.VMEM((128, D), F32),
            pltpu.SemaphoreType.DMA,
        ],
    )(temb_h, temb_d, temb_s, apad, upd, ups)


def _sc_edge(tdst, tsrc, dsti, srci, zer):
    return pl.kernel(
        _sc_edge_body,
        mesh=_mesh(),
        out_type=[
            jax.ShapeDtypeStruct((2 * NP, SW), F32),
            jax.ShapeDtypeStruct((2 * NT * H,), F32),
        ],
        scratch_types=[
            pltpu.VMEM_SHARED((NP, SW), F32),
            pltpu.VMEM((CH,), jnp.int32),
            pltpu.VMEM((CH,), jnp.int32),
            pltpu.VMEM((CH,), jnp.int32),
            pltpu.VMEM((CH,), jnp.int32),
            pltpu.VMEM((CH, D), F32),
            pltpu.VMEM((CH, D), F32),
            pltpu.VMEM((CH, SW), F32),
            pltpu.VMEM((H,), F32),
            pltpu.SemaphoreType.DMA,
        ],
    )(tdst, tsrc, dsti, srci, zer)


def _mask_rows(i, x):
    rows = i * BLK + lax.broadcasted_iota(jnp.int32, (BLK, 1), 0)
    return jnp.where(rows < N, x, 0.0)


def _c_emb_body(emb_r, wnd_r, wns_r, bd_r, td_r, ts_r):
    e = emb_r[...]
    td_r[...] = jnp.dot(e, wnd_r[0], preferred_element_type=F32) + bd_r[0]
    ts_r[...] = jnp.dot(e, wns_r[0], preferred_element_type=F32)


def _c_emb(emb, wnd, wns, bd):
    return pl.pallas_call(
        _c_emb_body,
        grid=(2,),
        in_specs=[
            pl.BlockSpec((120, D), lambda c: (0, 0)),
            pl.BlockSpec((1, D, D), lambda c: (c, 0, 0)),
            pl.BlockSpec((1, D, D), lambda c: (c, 0, 0)),
            pl.BlockSpec((1, 1, D), lambda c: (c, 0, 0)),
        ],
        out_specs=[
            pl.BlockSpec((120, D), lambda c: (c, 0)),
            pl.BlockSpec((120, D), lambda c: (c, 0)),
        ],
        out_shape=[
            jax.ShapeDtypeStruct((240, D), F32),
            jax.ShapeDtypeStruct((240, D), F32),
        ],
    )(jnp.pad(emb, ((0, 2), (0, 0))), wnd, wns, bd.reshape(2, 1, D))


def _c_pos_body(pos_r, wpd_r, wps_r, ud_r, us_r):
    p = pos_r[...]
    ud_r[...] = jnp.dot(p, wpd_r[0], preferred_element_type=F32)
    us_r[...] = jnp.dot(p, wps_r[0], preferred_element_type=F32)


def _c_pos(pos_pad, wpd, wps):
    return pl.pallas_call(
        _c_pos_body,
        grid=(2, NB),
        in_specs=[
            pl.BlockSpec((BLK, 4), lambda c, i: (i, 0)),
            pl.BlockSpec((1, 4, D), lambda c, i: (c, 0, 0)),
            pl.BlockSpec((1, 4, D), lambda c, i: (c, 0, 0)),
        ],
        out_specs=[
            pl.BlockSpec((BLK, D), lambda c, i: (c * NB + i, 0)),
            pl.BlockSpec((BLK, D), lambda c, i: (c * NB + i, 0)),
        ],
        out_shape=[
            jax.ShapeDtypeStruct((2 * NP, D), F32),
            jax.ShapeDtypeStruct((2 * NP, D), F32),
        ],
    )(pos_pad, wpd, wps)


def _c_stats_body(h_r, s0_r, s1_r, cnt_r, out_r):
    i = pl.program_id(0)
    h = _mask_rows(i, h_r[...])
    S = jnp.concatenate([s0_r[...], s1_r[...]], axis=1)
    cnt = cnt_r[:, :1]
    st = jnp.stack([
        jnp.sum(h, axis=0),
        jnp.sum(h * h, axis=0),
        jnp.sum(h * S, axis=0),
        jnp.sum(h * cnt, axis=0),
        jnp.sum(S, axis=0),
        jnp.sum(S * S, axis=0),
        jnp.sum(S * cnt, axis=0),
        jnp.zeros((D,), F32) + jnp.sum(cnt * cnt),
    ])

    @pl.when(i == 0)
    def _():
        out_r[...] = st

    @pl.when(i > 0)
    def _():
        out_r[...] = out_r[...] + st


def _c_stats(h, s_arr, cnt):
    return pl.pallas_call(
        _c_stats_body,
        grid=(NB,),
        in_specs=[
            pl.BlockSpec((BLK, D), lambda i: (i, 0)),
            pl.BlockSpec((BLK, SW), lambda i: (i, 0)),
            pl.BlockSpec((BLK, SW), lambda i: (NB + i, 0)),
            pl.BlockSpec((BLK, SW), lambda i: (i, 0)),
        ],
        out_specs=pl.BlockSpec((8, D), lambda i: (0, 0)),
        out_shape=jax.ShapeDtypeStruct((8, D), F32),
    )(h, s_arr, s_arr, cnt)


def _coeffs(sums, m2, gm, bm, gu, bu, go, bo):
    Sh, Sh2, ShS, Shc, SS, SS2, SSc = (sums[k] for k in range(7))
    Cq = sums[7, 0]
    Ef = float(E)
    Nf = float(N)
    M2 = m2
    mu_m = SS / Ef
    vm = M2 / Ef - mu_m * mu_m
    k1 = gm / jnp.sqrt(vm + EPS)
    k2 = bm - mu_m * k1
    mu_a = (k1 * SS + k2 * Ef) / Nf
    Ea2 = (k1 * k1 * SS2 + 2.0 * k1 * k2 * SSc + k2 * k2 * Cq) / Nf
    va = Ea2 - mu_a * mu_a
    p1 = gu / jnp.sqrt(va + EPS)
    p2 = bu - mu_a * p1
    q1 = k1 * p1
    q2 = k2 * p1
    q3 = p2
    mu_u = (Sh + q1 * SS + q2 * Ef + Nf * q3) / Nf
    Eu2 = (Sh2 + q1 * q1 * SS2 + q2 * q2 * Cq + Nf * q3 * q3
           + 2.0 * q1 * ShS + 2.0 * q2 * Shc + 2.0 * q3 * Sh
           + 2.0 * q1 * q2 * SSc + 2.0 * q1 * q3 * SS + 2.0 * q2 * q3 * Ef) / Nf
    vu = Eu2 - mu_u * mu_u
    r1 = go / jnp.sqrt(vu + EPS)
    r2 = bo - mu_u * r1
    A = r1
    B = q1 * r1
    C = q2 * r1
    Dc = q3 * r1 + r2
    return A, B, C, Dc


def _c_update_body(h_r, s0_r, s1_r, cnt_r, sums_r, m2_r, gm_r, bm_r, gu_r,
                   bu_r, go_r, bo_r, out_r):
    i = pl.program_id(0)
    m2p = jnp.sum(m2_r[...], axis=1)
    m2 = jnp.concatenate([m2p[0], m2p[1]], axis=0)
    A, B, C, Dc = _coeffs(sums_r[...], m2, gm_r[0], bm_r[0], gu_r[0],
                          bu_r[0], go_r[0], bo_r[0])
    h = _mask_rows(i, h_r[...])
    S = jnp.concatenate([s0_r[...], s1_r[...]], axis=1)
    cnt = cnt_r[:, :1]
    u = h * A + S * B + cnt * C + Dc
    hn = jnp.maximum(u, 0.0) + jnp.log1p(jnp.exp(-jnp.abs(u)))
    out_r[...] = _mask_rows(i, hn)


def _c_update(h, s_arr, cnt, sums, m2, gm, bm, gu, bu, go, bo):
    vec = lambda: pl.BlockSpec((1, D), lambda i: (0, 0))
    return pl.pallas_call(
        _c_update_body,
        grid=(NB,),
        in_specs=[
            pl.BlockSpec((BLK, D), lambda i: (i, 0)),
            pl.BlockSpec((BLK, SW), lambda i: (i, 0)),
            pl.BlockSpec((BLK, SW), lambda i: (NB + i, 0)),
            pl.BlockSpec((BLK, SW), lambda i: (i, 0)),
            pl.BlockSpec((8, D), lambda i: (0, 0)),
            pl.BlockSpec((2, NT, H), lambda i: (0, 0, 0)),
            vec(), vec(), vec(), vec(), vec(), vec(),
        ],
        out_specs=pl.BlockSpec((BLK, D), lambda i: (i, 0)),
        out_shape=jax.ShapeDtypeStruct((NP, D), F32),
    )(h, s_arr, s_arr, cnt, sums, m2, gm.reshape(1, D), bm.reshape(1, D),
      gu.reshape(1, D), bu.reshape(1, D), go.reshape(1, D), bo.reshape(1, D))


def _c_tables_body(h_r, pos_r, wnd_r, wns_r, wpd_r, wps_r, bd_r, td_r, ts_r):
    h = h_r[...]
    p = pos_r[...]
    td_r[...] = (jnp.dot(h, wnd_r[0], preferred_element_type=F32)
                 + jnp.dot(p, wpd_r[0], preferred_element_type=F32) + bd_r[0])
    ts_r[...] = (jnp.dot(h, wns_r[0], preferred_element_type=F32)
                 + jnp.dot(p, wps_r[0], preferred_element_type=F32))


def _c_tables(h, pos_pad, wnd, wns, wpd, wps, bd):
    return pl.pallas_call(
        _c_tables_body,
        grid=(2, NB),
        in_specs=[
            pl.BlockSpec((BLK, D), lambda c, i: (i, 0)),
            pl.BlockSpec((BLK, 4), lambda c, i: (i, 0)),
            pl.BlockSpec((1, D, D), lambda c, i: (c, 0, 0)),
            pl.BlockSpec((1, D, D), lambda c, i: (c, 0, 0)),
            pl.BlockSpec((1, 4, D), lambda c, i: (c, 0, 0)),
            pl.BlockSpec((1, 4, D), lambda c, i: (c, 0, 0)),
            pl.BlockSpec((1, 1, D), lambda c, i: (c, 0, 0)),
        ],
        out_specs=[
            pl.BlockSpec((BLK, D), lambda c, i: (c * NB + i, 0)),
            pl.BlockSpec((BLK, D), lambda c, i: (c * NB + i, 0)),
        ],
        out_shape=[
            jax.ShapeDtypeStruct((2 * NP, D), F32),
            jax.ShapeDtypeStruct((2 * NP, D), F32),
        ],
    )(h, pos_pad, wnd, wns, wpd, wps, bd.reshape(2, 1, D))


def _c_final_body(h_r, w1t_r, b1_r, w2t_r, b2_r, out_r, acc_r):
    i = pl.program_id(0)
    cs = jnp.sum(h_r[...], axis=0, keepdims=True)

    @pl.when(i == 0)
    def _():
        acc_r[...] = cs

    @pl.when(i > 0)
    def _():
        acc_r[...] = acc_r[...] + cs

    @pl.when(i == NB - 1)
    def _():
        hp = acc_r[...] / float(N)
        pre = jnp.dot(hp, w1t_r[...], preferred_element_type=F32) + b1_r[...]
        hid = jnp.maximum(pre, 0.0) + jnp.log1p(jnp.exp(-jnp.abs(pre)))
        out_r[...] = jnp.dot(hid, w2t_r[...], preferred_element_type=F32) + b2_r[...]


def _c_final(h, w1t, b1, w2t, b2):
    return pl.pallas_call(
        _c_final_body,
        grid=(NB,),
        in_specs=[
            pl.BlockSpec((BLK, D), lambda i: (i, 0)),
            pl.BlockSpec((D, 2 * D), lambda i: (0, 0)),
            pl.BlockSpec((1, 2 * D), lambda i: (0, 0)),
            pl.BlockSpec((2 * D, 1), lambda i: (0, 0)),
            pl.BlockSpec((1, 1), lambda i: (0, 0)),
        ],
        out_specs=pl.BlockSpec((1, 1), lambda i: (0, 0)),
        out_shape=jax.ShapeDtypeStruct((1, 1), F32),
        scratch_shapes=[pltpu.VMEM((1, D), F32)],
    )(h, w1t, b1, w2t, b2)


def kernel(pos, atomic_numbers, edge_index, emb, lin_f_W, lin_f_b, lin_s_W,
           lin_s_b, bn_msg_g, bn_msg_b, bn_upd_g, bn_upd_b, bn_out_g,
           bn_out_b, mlp_W1, mlp_b1, mlp_W2, mlp_b2):
    src = edge_index[0].astype(jnp.int32)
    dst = edge_index[1].astype(jnp.int32)
    a_pad = jnp.pad(atomic_numbers.astype(jnp.int32), (0, NP - N))
    pos_pad = jnp.pad(pos.astype(F32), ((0, NP - N), (0, 1)))
    zer = jnp.zeros((NP, SW), F32)

    wnd, wns, wpd, wps, bdst = [], [], [], [], []
    for i in range(L):
        Wf, Ws = lin_f_W[i], lin_s_W[i]
        Wfd, Wfs, Wfe = Wf[:, :D].T, Wf[:, D:2 * D].T, Wf[:, 2 * D:].T
        Wsd, Wss, Wse = Ws[:, :D].T, Ws[:, D:2 * D].T, Ws[:, 2 * D:].T
        halves = lambda Wa, Wb: jnp.stack([
            jnp.concatenate([Wa[:, :H], Wb[:, :H]], axis=1),
            jnp.concatenate([Wa[:, H:], Wb[:, H:]], axis=1)])
        wnd.append(halves(Wfd, Wsd))
        wns.append(halves(Wfs, Wss))
        pad3 = lambda M: jnp.concatenate([M, jnp.zeros((1, D), F32)], axis=0)
        wpd.append(jnp.stack([
            pad3(jnp.concatenate([-Wfe[:, :H], -Wse[:, :H]], axis=1)),
            pad3(jnp.concatenate([-Wfe[:, H:], -Wse[:, H:]], axis=1))]))
        wps.append(-wpd[i])
        bf, bs = lin_f_b[i], lin_s_b[i]
        bdst.append(jnp.stack([
            jnp.concatenate([bf[:H], bs[:H]]),
            jnp.concatenate([bf[H:], bs[H:]])]))

    temb_d, temb_s = _c_emb(emb.astype(F32), wnd[0], wns[0], bdst[0])
    upd, ups = _c_pos(pos_pad, wpd[0], wps[0])
    h, tdst, tsrc = _sc_init(emb.astype(F32), temb_d, temb_s, a_pad, upd, ups)
    (cnt,) = _sc_count(dst, zer)

    for i in range(L):
        s_arr, m2 = _sc_edge(tdst, tsrc, dst, src, zer)
        m2 = m2.reshape(2, NT, H)
        sums = _c_stats(h, s_arr, cnt)
        h = _c_update(h, s_arr, cnt, sums, m2, bn_msg_g[i], bn_msg_b[i],
                      bn_upd_g[i], bn_upd_b[i], bn_out_g[i], bn_out_b[i])
        if i < L - 1:
            tdst, tsrc = _c_tables(h, pos_pad, wnd[i + 1], wns[i + 1],
                                   wpd[i + 1], wps[i + 1], bdst[i + 1])

    out = _c_final(h, mlp_W1.T.astype(F32), mlp_b1.reshape(1, 2 * D),
                   mlp_W2.T.astype(F32), mlp_b2.reshape(1, 1))
    return out.reshape(1)


# speedup vs baseline: 1.1371x; 1.1371x over previous
import jax
import jax.numpy as jnp
from jax import lax
from jax.experimental import pallas as pl
from jax.experimental.pallas import tpu as pltpu
from jax.experimental.pallas import tpu_sc as plsc

N = 10000
E = 160000
D = 256
H = 128
L = 3
NP = 10240
BLK = 512
NB = NP // BLK
SW = 128
NT = 16
NPT = NP // NT
EPT = E // NT
CH = 40
NCH = EPT // CH
EPS = 1e-5
F32 = jnp.float32

def _mesh():
    return plsc.VectorSubcoreMesh(core_axis_name="c", subcore_axis_name="s",
                                  num_cores=2, num_subcores=NT)


def _sc_init_body(temb_h, temb_d, temb_s, apad, upd, ups,
                  h0, t0d, t0s,
                  aidx, aadj, rows, urows, sem):
    c = lax.axis_index("c")
    s = lax.axis_index("s")
    nb = s * NPT

    def _add_rows(r, _):
        for g in range(D // 16):
            sl = pl.ds(g * 16, 16)
            rows[r, sl] = rows[r, sl] + urows[r, sl]
        return 0

    def chunk(j, _):
        off = nb + j * 128
        pltpu.sync_copy(apad.at[pl.ds(off, 128)], aidx)

        @pl.when(c == 0)
        def _():
            pltpu.async_copy(temb_h.at[aidx], rows, sem).wait()
            pltpu.sync_copy(rows, h0.at[pl.ds(off, 128)])

        for g in range(8):
            sl = pl.ds(g * 16, 16)
            aadj[sl] = aidx[sl] + c * 120

        pltpu.async_copy(temb_d.at[aadj], rows, sem).wait()
        pltpu.sync_copy(upd.at[pl.ds(c * NP + off, 128)], urows)
        lax.fori_loop(0, 128, _add_rows, 0)
        pltpu.sync_copy(rows, t0d.at[pl.ds(c * NP + off, 128)])

        pltpu.async_copy(temb_s.at[aadj], rows, sem).wait()
        pltpu.sync_copy(ups.at[pl.ds(c * NP + off, 128)], urows)
        lax.fori_loop(0, 128, _add_rows, 0)
        pltpu.sync_copy(rows, t0s.at[pl.ds(c * NP + off, 128)])
        return 0

    lax.fori_loop(0, NPT // 128, chunk, 0)


def _sc_edge_body(tdst, tsrc, ep, zer,
                  s_out, m2_out,
                  stab, pk0, pk1, dr0, dr1, da0, da1, sa0, sa1,
                  gd0, gd1, gs0, gs1, mb, acc, sem):
    c = lax.axis_index("c")
    s = lax.axis_index("s")

    pltpu.sync_copy(zer.at[pl.ds(s * NPT, NPT)], stab.at[pl.ds(s * NPT, NPT)])
    zv = jnp.zeros((16,), F32)
    for g in range(8):
        acc[pl.ds(g * 16, 16)] = zv
    plsc.subcore_barrier()

    cbase = s * NCH
    coff = c * NP
    c1, c2, c3, c4, c5 = (1.0 / 3, 1.0 / 5, 1.0 / 7, 1.0 / 9, 1.0 / 11)

    def load_idx(pk, dr, da, sa, gchunk):
        pltpu.sync_copy(ep.at[pl.ds(gchunk, 1)], pk)
        for o in (0, 16, 24):
            sl = pl.ds(o, 16)
            v = pk[0, sl]
            dr[sl] = v
            da[sl] = v + coff
            sa[sl] = pk[0, pl.ds(CH + o, 16)] + coff

    def issue(da, sa, gd, gs):
        pltpu.async_copy(tdst.at[da], gd, sem)
        pltpu.async_copy(tsrc.at[sa], gs, sem)

    def wait(da, sa, gd, gs):
        pltpu.make_async_copy(tdst.at[da], gd, sem).wait()
        pltpu.make_async_copy(tsrc.at[sa], gs, sem).wait()

    def compute_scatter(gd, gs, dr):
        def edge(e, _):
            for g in range(8):
                fo = pl.ds(g * 16, 16)
                so = pl.ds(H + g * 16, 16)
                f = gd[e, fo] + gs[e, fo]
                sv = gd[e, so] + gs[e, so]
                sig = 1.0 / (1.0 + jnp.exp(-f))
                t = jnp.exp(-jnp.abs(sv))
                z = t / (2.0 + t)
                z2 = z * z
                l1p = 2.0 * z * (1.0 + z2 * (c1 + z2 * (c2 + z2 * (c3 + z2 * (c4 + z2 * c5)))))
                m = sig * (jnp.maximum(sv, 0.0) + l1p)
                mb[e, fo] = m
                acc[fo] = acc[fo] + m * m
            return 0

        lax.fori_loop(0, CH, edge, 0)
        pltpu.sync_copy(mb, stab.at[dr], add=True)

    load_idx(pk0, dr0, da0, sa0, cbase)
    issue(da0, sa0, gd0, gs0)
    load_idx(pk1, dr1, da1, sa1, cbase + 1)

    def pair(u, _):
        a = cbase + 2 * u
        wait(da0, sa0, gd0, gs0)
        issue(da1, sa1, gd1, gs1)
        compute_scatter(gd0, gs0, dr0)

        @pl.when(2 * u + 2 < NCH)
        def _():
            load_idx(pk0, dr0, da0, sa0, a + 2)

        wait(da1, sa1, gd1, gs1)

        @pl.when(2 * u + 2 < NCH)
        def _():
            issue(da0, sa0, gd0, gs0)

        compute_scatter(gd1, gs1, dr1)

        @pl.when(2 * u + 3 < NCH)
        def _():
            load_idx(pk1, dr1, da1, sa1, a + 3)

        return 0

    lax.fori_loop(0, NCH // 2, pair, 0)
    plsc.subcore_barrier()

    pltpu.sync_copy(stab.at[pl.ds(s * NPT, NPT)],
                    s_out.at[pl.ds(coff + s * NPT, NPT)])
    pltpu.sync_copy(acc, m2_out.at[pl.ds((c * NT + s) * H, H)])


def _sc_count_body(dsti, zer, cnt_out, ctab, idr, ones, sem):
    c = lax.axis_index("c")
    s = lax.axis_index("s")

    @pl.when(c == 0)
    def _():
        pltpu.sync_copy(zer.at[pl.ds(s * NPT, NPT)],
                        ctab.at[pl.ds(s * NPT, NPT)])
        onev = jnp.where(lax.iota(jnp.int32, 16) == 0, 1.0, 0.0).astype(F32)
        zv = jnp.zeros((16,), F32)

        def _initrow(e, _):
            ones[e, pl.ds(0, 16)] = onev
            for g in range(1, 8):
                ones[e, pl.ds(g * 16, 16)] = zv
            return 0

        lax.fori_loop(0, CH, _initrow, 0)
        plsc.subcore_barrier()
        base = s * EPT

        def chunk(j, _):
            pltpu.sync_copy(dsti.at[pl.ds(base + j * CH, CH)], idr)
            pltpu.sync_copy(ones, ctab.at[idr], add=True)
            return 0

        lax.fori_loop(0, NCH, chunk, 0)
        plsc.subcore_barrier()
        pltpu.sync_copy(ctab.at[pl.ds(s * NPT, NPT)],
                        cnt_out.at[pl.ds(s * NPT, NPT)])


def _sc_count(dsti, zer):
    return pl.kernel(
        _sc_count_body,
        mesh=_mesh(),
        out_type=[jax.ShapeDtypeStruct((NP, SW), F32)],
        scratch_types=[
            pltpu.VMEM_SHARED((NP, SW), F32),
            pltpu.VMEM((CH,), jnp.int32),
            pltpu.VMEM((CH, SW), F32),
            pltpu.SemaphoreType.DMA,
        ],
    )(dsti, zer)


def _sc_init(temb_h, temb_d, temb_s, apad, upd, ups):
    return pl.kernel(
        _sc_init_body,
        mesh=_mesh(),
        out_type=[
            jax.ShapeDtypeStruct((NP, D), F32),
            jax.ShapeDtypeStruct((2 * NP, D), F32),
            jax.ShapeDtypeStruct((2 * NP, D), F32),
        ],
        scratch_types=[
            pltpu.VMEM((128,), jnp.int32),
            pltpu.VMEM((128,), jnp.int32),
            pltpu.VMEM((128, D), F32),
            pltpu.VMEM((128, D), F32),
            pltpu.SemaphoreType.DMA,
        ],
    )(temb_h, temb_d, temb_s, apad, upd, ups)


def _sc_edge(tdst, tsrc, ep, zer):
    idx32 = lambda: pltpu.VMEM((CH,), jnp.int32)
    gbuf = lambda: pltpu.VMEM((CH, D), F32)
    return pl.kernel(
        _sc_edge_body,
        mesh=_mesh(),
        out_type=[
            jax.ShapeDtypeStruct((2 * NP, SW), F32),
            jax.ShapeDtypeStruct((2 * NT * H,), F32),
        ],
        scratch_types=[
            pltpu.VMEM_SHARED((NP, SW), F32),
            pltpu.VMEM((1, 2 * CH), jnp.int32),
            pltpu.VMEM((1, 2 * CH), jnp.int32),
            idx32(), idx32(), idx32(), idx32(), idx32(), idx32(),
            gbuf(), gbuf(), gbuf(), gbuf(),
            pltpu.VMEM((CH, SW), F32),
            pltpu.VMEM((H,), F32),
            pltpu.SemaphoreType.DMA,
        ],
    )(tdst, tsrc, ep, zer)


def _mask_rows(i, x):
    rows = i * BLK + lax.broadcasted_iota(jnp.int32, (BLK, 1), 0)
    return jnp.where(rows < N, x, 0.0)


def _c_emb_body(emb_r, wnd_r, wns_r, bd_r, td_r, ts_r):
    e = emb_r[...]
    td_r[...] = jnp.dot(e, wnd_r[0], preferred_element_type=F32) + bd_r[0]
    ts_r[...] = jnp.dot(e, wns_r[0], preferred_element_type=F32)


def _c_emb(emb, wnd, wns, bd):
    return pl.pallas_call(
        _c_emb_body,
        grid=(2,),
        in_specs=[
            pl.BlockSpec((120, D), lambda c: (0, 0)),
            pl.BlockSpec((1, D, D), lambda c: (c, 0, 0)),
            pl.BlockSpec((1, D, D), lambda c: (c, 0, 0)),
            pl.BlockSpec((1, 1, D), lambda c: (c, 0, 0)),
        ],
        out_specs=[
            pl.BlockSpec((120, D), lambda c: (c, 0)),
            pl.BlockSpec((120, D), lambda c: (c, 0)),
        ],
        out_shape=[
            jax.ShapeDtypeStruct((240, D), F32),
            jax.ShapeDtypeStruct((240, D), F32),
        ],
    )(jnp.pad(emb, ((0, 2), (0, 0))), wnd, wns, bd.reshape(2, 1, D))


def _c_pos_body(pos_r, wpd_r, wps_r, ud_r, us_r):
    p = pos_r[...]
    ud_r[...] = jnp.dot(p, wpd_r[0], preferred_element_type=F32)
    us_r[...] = jnp.dot(p, wps_r[0], preferred_element_type=F32)


def _c_pos(pos_pad, wpd, wps):
    return pl.pallas_call(
        _c_pos_body,
        grid=(2, NB),
        in_specs=[
            pl.BlockSpec((BLK, 4), lambda c, i: (i, 0)),
            pl.BlockSpec((1, 4, D), lambda c, i: (c, 0, 0)),
            pl.BlockSpec((1, 4, D), lambda c, i: (c, 0, 0)),
        ],
        out_specs=[
            pl.BlockSpec((BLK, D), lambda c, i: (c * NB + i, 0)),
            pl.BlockSpec((BLK, D), lambda c, i: (c * NB + i, 0)),
        ],
        out_shape=[
            jax.ShapeDtypeStruct((2 * NP, D), F32),
            jax.ShapeDtypeStruct((2 * NP, D), F32),
        ],
    )(pos_pad, wpd, wps)


def _c_stats_body(h_r, s0_r, s1_r, cnt_r, out_r):
    i = pl.program_id(0)
    h = _mask_rows(i, h_r[...])
    S = jnp.concatenate([s0_r[...], s1_r[...]], axis=1)
    cnt = cnt_r[:, :1]
    st = jnp.stack([
        jnp.sum(h, axis=0),
        jnp.sum(h * h, axis=0),
        jnp.sum(h * S, axis=0),
        jnp.sum(h * cnt, axis=0),
        jnp.sum(S, axis=0),
        jnp.sum(S * S, axis=0),
        jnp.sum(S * cnt, axis=0),
        jnp.zeros((D,), F32) + jnp.sum(cnt * cnt),
    ])

    @pl.when(i == 0)
    def _():
        out_r[...] = st

    @pl.when(i > 0)
    def _():
        out_r[...] = out_r[...] + st


def _c_stats(h, s_arr, cnt):
    return pl.pallas_call(
        _c_stats_body,
        grid=(NB,),
        in_specs=[
            pl.BlockSpec((BLK, D), lambda i: (i, 0)),
            pl.BlockSpec((BLK, SW), lambda i: (i, 0)),
            pl.BlockSpec((BLK, SW), lambda i: (NB + i, 0)),
            pl.BlockSpec((BLK, SW), lambda i: (i, 0)),
        ],
        out_specs=pl.BlockSpec((8, D), lambda i: (0, 0)),
        out_shape=jax.ShapeDtypeStruct((8, D), F32),
    )(h, s_arr, s_arr, cnt)


def _coeffs(sums, m2, gm, bm, gu, bu, go, bo):
    Sh, Sh2, ShS, Shc, SS, SS2, SSc = (sums[k] for k in range(7))
    Cq = sums[7, 0]
    Ef = float(E)
    Nf = float(N)
    M2 = m2
    mu_m = SS / Ef
    vm = M2 / Ef - mu_m * mu_m
    k1 = gm / jnp.sqrt(vm + EPS)
    k2 = bm - mu_m * k1
    mu_a = (k1 * SS + k2 * Ef) / Nf
    Ea2 = (k1 * k1 * SS2 + 2.0 * k1 * k2 * SSc + k2 * k2 * Cq) / Nf
    va = Ea2 - mu_a * mu_a
    p1 = gu / jnp.sqrt(va + EPS)
    p2 = bu - mu_a * p1
    q1 = k1 * p1
    q2 = k2 * p1
    q3 = p2
    mu_u = (Sh + q1 * SS + q2 * Ef + Nf * q3) / Nf
    Eu2 = (Sh2 + q1 * q1 * SS2 + q2 * q2 * Cq + Nf * q3 * q3
           + 2.0 * q1 * ShS + 2.0 * q2 * Shc + 2.0 * q3 * Sh
           + 2.0 * q1 * q2 * SSc + 2.0 * q1 * q3 * SS + 2.0 * q2 * q3 * Ef) / Nf
    vu = Eu2 - mu_u * mu_u
    r1 = go / jnp.sqrt(vu + EPS)
    r2 = bo - mu_u * r1
    A = r1
    B = q1 * r1
    C = q2 * r1
    Dc = q3 * r1 + r2
    return A, B, C, Dc


def _c_update_body(h_r, s0_r, s1_r, cnt_r, sums_r, m2_r, gm_r, bm_r, gu_r,
                   bu_r, go_r, bo_r, out_r):
    i = pl.program_id(0)
    m2p = jnp.sum(m2_r[...], axis=1)
    m2 = jnp.concatenate([m2p[0], m2p[1]], axis=0)
    A, B, C, Dc = _coeffs(sums_r[...], m2, gm_r[0], bm_r[0], gu_r[0],
                          bu_r[0], go_r[0], bo_r[0])
    h = _mask_rows(i, h_r[...])
    S = jnp.concatenate([s0_r[...], s1_r[...]], axis=1)
    cnt = cnt_r[:, :1]
    u = h * A + S * B + cnt * C + Dc
    hn = jnp.maximum(u, 0.0) + jnp.log1p(jnp.exp(-jnp.abs(u)))
    out_r[...] = _mask_rows(i, hn)


def _c_update(h, s_arr, cnt, sums, m2, gm, bm, gu, bu, go, bo):
    vec = lambda: pl.BlockSpec((1, D), lambda i: (0, 0))
    return pl.pallas_call(
        _c_update_body,
        grid=(NB,),
        in_specs=[
            pl.BlockSpec((BLK, D), lambda i: (i, 0)),
            pl.BlockSpec((BLK, SW), lambda i: (i, 0)),
            pl.BlockSpec((BLK, SW), lambda i: (NB + i, 0)),
            pl.BlockSpec((BLK, SW), lambda i: (i, 0)),
            pl.BlockSpec((8, D), lambda i: (0, 0)),
            pl.BlockSpec((2, NT, H), lambda i: (0, 0, 0)),
            vec(), vec(), vec(), vec(), vec(), vec(),
        ],
        out_specs=pl.BlockSpec((BLK, D), lambda i: (i, 0)),
        out_shape=jax.ShapeDtypeStruct((NP, D), F32),
    )(h, s_arr, s_arr, cnt, sums, m2, gm.reshape(1, D), bm.reshape(1, D),
      gu.reshape(1, D), bu.reshape(1, D), go.reshape(1, D), bo.reshape(1, D))


def _c_tables_body(h_r, pos_r, wnd_r, wns_r, wpd_r, wps_r, bd_r, td_r, ts_r):
    h = h_r[...]
    p = pos_r[...]
    td_r[...] = (jnp.dot(h, wnd_r[0], preferred_element_type=F32)
                 + jnp.dot(p, wpd_r[0], preferred_element_type=F32) + bd_r[0])
    ts_r[...] = (jnp.dot(h, wns_r[0], preferred_element_type=F32)
                 + jnp.dot(p, wps_r[0], preferred_element_type=F32))


def _c_tables(h, pos_pad, wnd, wns, wpd, wps, bd):
    return pl.pallas_call(
        _c_tables_body,
        grid=(2, NB),
        in_specs=[
            pl.BlockSpec((BLK, D), lambda c, i: (i, 0)),
            pl.BlockSpec((BLK, 4), lambda c, i: (i, 0)),
            pl.BlockSpec((1, D, D), lambda c, i: (c, 0, 0)),
            pl.BlockSpec((1, D, D), lambda c, i: (c, 0, 0)),
            pl.BlockSpec((1, 4, D), lambda c, i: (c, 0, 0)),
            pl.BlockSpec((1, 4, D), lambda c, i: (c, 0, 0)),
            pl.BlockSpec((1, 1, D), lambda c, i: (c, 0, 0)),
        ],
        out_specs=[
            pl.BlockSpec((BLK, D), lambda c, i: (c * NB + i, 0)),
            pl.BlockSpec((BLK, D), lambda c, i: (c * NB + i, 0)),
        ],
        out_shape=[
            jax.ShapeDtypeStruct((2 * NP, D), F32),
            jax.ShapeDtypeStruct((2 * NP, D), F32),
        ],
    )(h, pos_pad, wnd, wns, wpd, wps, bd.reshape(2, 1, D))


def _c_final_body(h_r, w1t_r, b1_r, w2t_r, b2_r, out_r, acc_r):
    i = pl.program_id(0)
    cs = jnp.sum(h_r[...], axis=0, keepdims=True)

    @pl.when(i == 0)
    def _():
        acc_r[...] = cs

    @pl.when(i > 0)
    def _():
        acc_r[...] = acc_r[...] + cs

    @pl.when(i == NB - 1)
    def _():
        hp = acc_r[...] / float(N)
        pre = jnp.dot(hp, w1t_r[...], preferred_element_type=F32) + b1_r[...]
        hid = jnp.maximum(pre, 0.0) + jnp.log1p(jnp.exp(-jnp.abs(pre)))
        out_r[...] = jnp.dot(hid, w2t_r[...], preferred_element_type=F32) + b2_r[...]


def _c_final(h, w1t, b1, w2t, b2):
    return pl.pallas_call(
        _c_final_body,
        grid=(NB,),
        in_specs=[
            pl.BlockSpec((BLK, D), lambda i: (i, 0)),
            pl.BlockSpec((D, 2 * D), lambda i: (0, 0)),
            pl.BlockSpec((1, 2 * D), lambda i: (0, 0)),
            pl.BlockSpec((2 * D, 1), lambda i: (0, 0)),
            pl.BlockSpec((1, 1), lambda i: (0, 0)),
        ],
        out_specs=pl.BlockSpec((1, 1), lambda i: (0, 0)),
        out_shape=jax.ShapeDtypeStruct((1, 1), F32),
        scratch_shapes=[pltpu.VMEM((1, D), F32)],
    )(h, w1t, b1, w2t, b2)


def kernel(pos, atomic_numbers, edge_index, emb, lin_f_W, lin_f_b, lin_s_W,
           lin_s_b, bn_msg_g, bn_msg_b, bn_upd_g, bn_upd_b, bn_out_g,
           bn_out_b, mlp_W1, mlp_b1, mlp_W2, mlp_b2):
    src = edge_index[0].astype(jnp.int32)
    dst = edge_index[1].astype(jnp.int32)
    ep = jnp.concatenate([dst.reshape(-1, CH), src.reshape(-1, CH)], axis=1)
    a_pad = jnp.pad(atomic_numbers.astype(jnp.int32), (0, NP - N))
    pos_pad = jnp.pad(pos.astype(F32), ((0, NP - N), (0, 1)))
    zer = jnp.zeros((NP, SW), F32)

    wnd, wns, wpd, wps, bdst = [], [], [], [], []
    for i in range(L):
        Wf, Ws = lin_f_W[i], lin_s_W[i]
        Wfd, Wfs, Wfe = Wf[:, :D].T, Wf[:, D:2 * D].T, Wf[:, 2 * D:].T
        Wsd, Wss, Wse = Ws[:, :D].T, Ws[:, D:2 * D].T, Ws[:, 2 * D:].T
        halves = lambda Wa, Wb: jnp.stack([
            jnp.concatenate([Wa[:, :H], Wb[:, :H]], axis=1),
            jnp.concatenate([Wa[:, H:], Wb[:, H:]], axis=1)])
        wnd.append(halves(Wfd, Wsd))
        wns.append(halves(Wfs, Wss))
        pad3 = lambda M: jnp.concatenate([M, jnp.zeros((1, D), F32)], axis=0)
        wpd.append(jnp.stack([
            pad3(jnp.concatenate([-Wfe[:, :H], -Wse[:, :H]], axis=1)),
            pad3(jnp.concatenate([-Wfe[:, H:], -Wse[:, H:]], axis=1))]))
        wps.append(-wpd[i])
        bf, bs = lin_f_b[i], lin_s_b[i]
        bdst.append(jnp.stack([
            jnp.concatenate([bf[:H], bs[:H]]),
            jnp.concatenate([bf[H:], bs[H:]])]))

    temb_d, temb_s = _c_emb(emb.astype(F32), wnd[0], wns[0], bdst[0])
    upd, ups = _c_pos(pos_pad, wpd[0], wps[0])
    h, tdst, tsrc = _sc_init(emb.astype(F32), temb_d, temb_s, a_pad, upd, ups)
    (cnt,) = _sc_count(dst, zer)

    for i in range(L):
        s_arr, m2 = _sc_edge(tdst, tsrc, ep, zer)
        m2 = m2.reshape(2, NT, H)
        sums = _c_stats(h, s_arr, cnt)
        h = _c_update(h, s_arr, cnt, sums, m2, bn_msg_g[i], bn_msg_b[i],
                      bn_upd_g[i], bn_upd_b[i], bn_out_g[i], bn_out_b[i])
        if i < L - 1:
            tdst, tsrc = _c_tables(h, pos_pad, wnd[i + 1], wns[i + 1],
                                   wpd[i + 1], wps[i + 1], bdst[i + 1])

    out = _c_final(h, mlp_W1.T.astype(F32), mlp_b1.reshape(1, 2 * D),
                   mlp_W2.T.astype(F32), mlp_b2.reshape(1, 1))
    return out.reshape(1)


# speedup vs baseline: 1.2379x; 1.0886x over previous
import jax
import jax.numpy as jnp
from jax import lax
from jax.experimental import pallas as pl
from jax.experimental.pallas import tpu as pltpu
from jax.experimental.pallas import tpu_sc as plsc

N = 10000
E = 160000
D = 256
H = 128
L = 3
NP = 10240
BLK = 512
NB = NP // BLK
SW = 128
NT = 16
NPT = NP // NT
EPT = E // NT
CH = 40
NCH = EPT // CH
EPS = 1e-5
F32 = jnp.float32

def _mesh():
    return plsc.VectorSubcoreMesh(core_axis_name="c", subcore_axis_name="s",
                                  num_cores=2, num_subcores=NT)


def _sc_init_body(temb_h, temb_d, temb_s, apad, upd, ups,
                  h0, t0d, t0s,
                  aidx, aadj, rows, urows, sem):
    c = lax.axis_index("c")
    s = lax.axis_index("s")
    nb = s * NPT

    def _add_rows(r, _):
        for g in range(D // 16):
            sl = pl.ds(g * 16, 16)
            rows[r, sl] = rows[r, sl] + urows[r, sl]
        return 0

    def chunk(j, _):
        off = nb + j * 128
        pltpu.sync_copy(apad.at[pl.ds(off, 128)], aidx)

        @pl.when(c == 0)
        def _():
            pltpu.async_copy(temb_h.at[aidx], rows, sem).wait()
            pltpu.sync_copy(rows, h0.at[pl.ds(off, 128)])

        for g in range(8):
            sl = pl.ds(g * 16, 16)
            aadj[sl] = aidx[sl] + c * 120

        pltpu.async_copy(temb_d.at[aadj], rows, sem).wait()
        pltpu.sync_copy(upd.at[pl.ds(c * NP + off, 128)], urows)
        lax.fori_loop(0, 128, _add_rows, 0)
        pltpu.sync_copy(rows, t0d.at[pl.ds(c * NP + off, 128)])

        pltpu.async_copy(temb_s.at[aadj], rows, sem).wait()
        pltpu.sync_copy(ups.at[pl.ds(c * NP + off, 128)], urows)
        lax.fori_loop(0, 128, _add_rows, 0)
        pltpu.sync_copy(rows, t0s.at[pl.ds(c * NP + off, 128)])
        return 0

    lax.fori_loop(0, NPT // 128, chunk, 0)


def _sc_edge_body(tdst, tsrc, ep, zer,
                  s_out, m2_out,
                  stab, pk0, pk1, dr0, dr1, da0, da1, sa0, sa1,
                  gd0, gd1, gs0, gs1, mb, acc, sem):
    c = lax.axis_index("c")
    s = lax.axis_index("s")

    pltpu.sync_copy(zer.at[pl.ds(s * NPT, NPT)], stab.at[pl.ds(s * NPT, NPT)])
    zv = jnp.zeros((16,), F32)
    for g in range(8):
        acc[pl.ds(g * 16, 16)] = zv
    plsc.subcore_barrier()

    cbase = s * NCH
    coff = c * NP
    c1, c2, c3, c4, c5 = (1.0 / 3, 1.0 / 5, 1.0 / 7, 1.0 / 9, 1.0 / 11)

    def load_idx(pk, dr, da, sa, gchunk):
        pltpu.sync_copy(ep.at[pl.ds(gchunk, 1)], pk)
        for o in (0, 16, 24):
            sl = pl.ds(o, 16)
            v = pk[0, sl]
            dr[sl] = v
            da[sl] = v + coff
            sa[sl] = pk[0, pl.ds(CH + o, 16)] + coff

    def issue(da, sa, gd, gs):
        pltpu.async_copy(tdst.at[da], gd, sem)
        pltpu.async_copy(tsrc.at[sa], gs, sem)

    def wait(da, sa, gd, gs):
        pltpu.make_async_copy(tdst.at[da], gd, sem).wait()
        pltpu.make_async_copy(tsrc.at[sa], gs, sem).wait()

    zero16 = jnp.zeros((16,), F32)

    def compute_scatter(gd, gs, dr):
        @plsc.parallel_loop(0, CH, unroll=2, carry=(zero16,) * 8)
        def a8(e, a8):
            out = []
            for g in range(8):
                fo = pl.ds(g * 16, 16)
                so = pl.ds(H + g * 16, 16)
                f = gd[e, fo] + gs[e, fo]
                sv = gd[e, so] + gs[e, so]
                sig = 1.0 / (1.0 + jnp.exp(-f))
                t = jnp.exp(-jnp.abs(sv))
                z = t / (2.0 + t)
                z2 = z * z
                l1p = 2.0 * z * (1.0 + z2 * (c1 + z2 * (c2 + z2 * (c3 + z2 * (c4 + z2 * c5)))))
                m = sig * (jnp.maximum(sv, 0.0) + l1p)
                mb[e, fo] = m
                out.append(a8[g] + m * m)
            return tuple(out)

        for g in range(8):
            fo = pl.ds(g * 16, 16)
            acc[fo] = acc[fo] + a8[g]
        pltpu.sync_copy(mb, stab.at[dr], add=True)

    load_idx(pk0, dr0, da0, sa0, cbase)
    issue(da0, sa0, gd0, gs0)
    load_idx(pk1, dr1, da1, sa1, cbase + 1)

    def pair(u, _):
        a = cbase + 2 * u
        wait(da0, sa0, gd0, gs0)
        issue(da1, sa1, gd1, gs1)
        compute_scatter(gd0, gs0, dr0)

        @pl.when(2 * u + 2 < NCH)
        def _():
            load_idx(pk0, dr0, da0, sa0, a + 2)

        wait(da1, sa1, gd1, gs1)

        @pl.when(2 * u + 2 < NCH)
        def _():
            issue(da0, sa0, gd0, gs0)

        compute_scatter(gd1, gs1, dr1)

        @pl.when(2 * u + 3 < NCH)
        def _():
            load_idx(pk1, dr1, da1, sa1, a + 3)

        return 0

    lax.fori_loop(0, NCH // 2, pair, 0)
    plsc.subcore_barrier()

    pltpu.sync_copy(stab.at[pl.ds(s * NPT, NPT)],
                    s_out.at[pl.ds(coff + s * NPT, NPT)])
    pltpu.sync_copy(acc, m2_out.at[pl.ds((c * NT + s) * H, H)])


def _sc_count_body(dsti, zer, cnt_out, ctab, idr, ones, sem):
    c = lax.axis_index("c")
    s = lax.axis_index("s")

    @pl.when(c == 0)
    def _():
        pltpu.sync_copy(zer.at[pl.ds(s * NPT, NPT)],
                        ctab.at[pl.ds(s * NPT, NPT)])
        onev = jnp.where(lax.iota(jnp.int32, 16) == 0, 1.0, 0.0).astype(F32)
        zv = jnp.zeros((16,), F32)

        def _initrow(e, _):
            ones[e, pl.ds(0, 16)] = onev
            for g in range(1, 8):
                ones[e, pl.ds(g * 16, 16)] = zv
            return 0

        lax.fori_loop(0, CH, _initrow, 0)
        plsc.subcore_barrier()
        base = s * EPT

        def chunk(j, _):
            pltpu.sync_copy(dsti.at[pl.ds(base + j * CH, CH)], idr)
            pltpu.sync_copy(ones, ctab.at[idr], add=True)
            return 0

        lax.fori_loop(0, NCH, chunk, 0)
        plsc.subcore_barrier()
        pltpu.sync_copy(ctab.at[pl.ds(s * NPT, NPT)],
                        cnt_out.at[pl.ds(s * NPT, NPT)])


def _sc_count(dsti, zer):
    return pl.kernel(
        _sc_count_body,
        mesh=_mesh(),
        out_type=[jax.ShapeDtypeStruct((NP, SW), F32)],
        scratch_types=[
            pltpu.VMEM_SHARED((NP, SW), F32),
            pltpu.VMEM((CH,), jnp.int32),
            pltpu.VMEM((CH, SW), F32),
            pltpu.SemaphoreType.DMA,
        ],
    )(dsti, zer)


def _sc_init(temb_h, temb_d, temb_s, apad, upd, ups):
    return pl.kernel(
        _sc_init_body,
        mesh=_mesh(),
        out_type=[
            jax.ShapeDtypeStruct((NP, D), F32),
            jax.ShapeDtypeStruct((2 * NP, D), F32),
            jax.ShapeDtypeStruct((2 * NP, D), F32),
        ],
        scratch_types=[
            pltpu.VMEM((128,), jnp.int32),
            pltpu.VMEM((128,), jnp.int32),
            pltpu.VMEM((128, D), F32),
            pltpu.VMEM((128, D), F32),
            pltpu.SemaphoreType.DMA,
        ],
    )(temb_h, temb_d, temb_s, apad, upd, ups)


def _sc_edge(tdst, tsrc, ep, zer):
    idx32 = lambda: pltpu.VMEM((CH,), jnp.int32)
    gbuf = lambda: pltpu.VMEM((CH, D), F32)
    return pl.kernel(
        _sc_edge_body,
        mesh=_mesh(),
        out_type=[
            jax.ShapeDtypeStruct((2 * NP, SW), F32),
            jax.ShapeDtypeStruct((2 * NT * H,), F32),
        ],
        scratch_types=[
            pltpu.VMEM_SHARED((NP, SW), F32),
            pltpu.VMEM((1, 2 * CH), jnp.int32),
            pltpu.VMEM((1, 2 * CH), jnp.int32),
            idx32(), idx32(), idx32(), idx32(), idx32(), idx32(),
            gbuf(), gbuf(), gbuf(), gbuf(),
            pltpu.VMEM((CH, SW), F32),
            pltpu.VMEM((H,), F32),
            pltpu.SemaphoreType.DMA,
        ],
    )(tdst, tsrc, ep, zer)


def _mask_rows(i, x):
    rows = i * BLK + lax.broadcasted_iota(jnp.int32, (BLK, 1), 0)
    return jnp.where(rows < N, x, 0.0)


def _c_emb_body(emb_r, wnd_r, wns_r, bd_r, td_r, ts_r):
    e = emb_r[...]
    td_r[...] = jnp.dot(e, wnd_r[0], preferred_element_type=F32) + bd_r[0]
    ts_r[...] = jnp.dot(e, wns_r[0], preferred_element_type=F32)


def _c_emb(emb, wnd, wns, bd):
    return pl.pallas_call(
        _c_emb_body,
        grid=(2,),
        in_specs=[
            pl.BlockSpec((120, D), lambda c: (0, 0)),
            pl.BlockSpec((1, D, D), lambda c: (c, 0, 0)),
            pl.BlockSpec((1, D, D), lambda c: (c, 0, 0)),
            pl.BlockSpec((1, 1, D), lambda c: (c, 0, 0)),
        ],
        out_specs=[
            pl.BlockSpec((120, D), lambda c: (c, 0)),
            pl.BlockSpec((120, D), lambda c: (c, 0)),
        ],
        out_shape=[
            jax.ShapeDtypeStruct((240, D), F32),
            jax.ShapeDtypeStruct((240, D), F32),
        ],
    )(jnp.pad(emb, ((0, 2), (0, 0))), wnd, wns, bd.reshape(2, 1, D))


def _c_pos_body(pos_r, wpd_r, wps_r, ud_r, us_r):
    p = pos_r[...]
    ud_r[...] = jnp.dot(p, wpd_r[0], preferred_element_type=F32)
    us_r[...] = jnp.dot(p, wps_r[0], preferred_element_type=F32)


def _c_pos(pos_pad, wpd, wps):
    return pl.pallas_call(
        _c_pos_body,
        grid=(2, NB),
        in_specs=[
            pl.BlockSpec((BLK, 4), lambda c, i: (i, 0)),
            pl.BlockSpec((1, 4, D), lambda c, i: (c, 0, 0)),
            pl.BlockSpec((1, 4, D), lambda c, i: (c, 0, 0)),
        ],
        out_specs=[
            pl.BlockSpec((BLK, D), lambda c, i: (c * NB + i, 0)),
            pl.BlockSpec((BLK, D), lambda c, i: (c * NB + i, 0)),
        ],
        out_shape=[
            jax.ShapeDtypeStruct((2 * NP, D), F32),
            jax.ShapeDtypeStruct((2 * NP, D), F32),
        ],
    )(pos_pad, wpd, wps)


def _c_stats_body(h_r, s0_r, s1_r, cnt_r, out_r):
    i = pl.program_id(0)
    h = _mask_rows(i, h_r[...])
    S = jnp.concatenate([s0_r[...], s1_r[...]], axis=1)
    cnt = cnt_r[:, :1]
    st = jnp.stack([
        jnp.sum(h, axis=0),
        jnp.sum(h * h, axis=0),
        jnp.sum(h * S, axis=0),
        jnp.sum(h * cnt, axis=0),
        jnp.sum(S, axis=0),
        jnp.sum(S * S, axis=0),
        jnp.sum(S * cnt, axis=0),
        jnp.zeros((D,), F32) + jnp.sum(cnt * cnt),
    ])

    @pl.when(i == 0)
    def _():
        out_r[...] = st

    @pl.when(i > 0)
    def _():
        out_r[...] = out_r[...] + st


def _c_stats(h, s_arr, cnt):
    return pl.pallas_call(
        _c_stats_body,
        grid=(NB,),
        in_specs=[
            pl.BlockSpec((BLK, D), lambda i: (i, 0)),
            pl.BlockSpec((BLK, SW), lambda i: (i, 0)),
            pl.BlockSpec((BLK, SW), lambda i: (NB + i, 0)),
            pl.BlockSpec((BLK, SW), lambda i: (i, 0)),
        ],
        out_specs=pl.BlockSpec((8, D), lambda i: (0, 0)),
        out_shape=jax.ShapeDtypeStruct((8, D), F32),
    )(h, s_arr, s_arr, cnt)


def _coeffs(sums, m2, gm, bm, gu, bu, go, bo):
    Sh, Sh2, ShS, Shc, SS, SS2, SSc = (sums[k] for k in range(7))
    Cq = sums[7, 0]
    Ef = float(E)
    Nf = float(N)
    M2 = m2
    mu_m = SS / Ef
    vm = M2 / Ef - mu_m * mu_m
    k1 = gm / jnp.sqrt(vm + EPS)
    k2 = bm - mu_m * k1
    mu_a = (k1 * SS + k2 * Ef) / Nf
    Ea2 = (k1 * k1 * SS2 + 2.0 * k1 * k2 * SSc + k2 * k2 * Cq) / Nf
    va = Ea2 - mu_a * mu_a
    p1 = gu / jnp.sqrt(va + EPS)
    p2 = bu - mu_a * p1
    q1 = k1 * p1
    q2 = k2 * p1
    q3 = p2
    mu_u = (Sh + q1 * SS + q2 * Ef + Nf * q3) / Nf
    Eu2 = (Sh2 + q1 * q1 * SS2 + q2 * q2 * Cq + Nf * q3 * q3
           + 2.0 * q1 * ShS + 2.0 * q2 * Shc + 2.0 * q3 * Sh
           + 2.0 * q1 * q2 * SSc + 2.0 * q1 * q3 * SS + 2.0 * q2 * q3 * Ef) / Nf
    vu = Eu2 - mu_u * mu_u
    r1 = go / jnp.sqrt(vu + EPS)
    r2 = bo - mu_u * r1
    A = r1
    B = q1 * r1
    C = q2 * r1
    Dc = q3 * r1 + r2
    return A, B, C, Dc


def _c_update_body(h_r, s0_r, s1_r, cnt_r, sums_r, m2_r, gm_r, bm_r, gu_r,
                   bu_r, go_r, bo_r, out_r):
    i = pl.program_id(0)
    m2p = jnp.sum(m2_r[...], axis=1)
    m2 = jnp.concatenate([m2p[0], m2p[1]], axis=0)
    A, B, C, Dc = _coeffs(sums_r[...], m2, gm_r[0], bm_r[0], gu_r[0],
                          bu_r[0], go_r[0], bo_r[0])
    h = _mask_rows(i, h_r[...])
    S = jnp.concatenate([s0_r[...], s1_r[...]], axis=1)
    cnt = cnt_r[:, :1]
    u = h * A + S * B + cnt * C + Dc
    hn = jnp.maximum(u, 0.0) + jnp.log1p(jnp.exp(-jnp.abs(u)))
    out_r[...] = _mask_rows(i, hn)


def _c_update(h, s_arr, cnt, sums, m2, gm, bm, gu, bu, go, bo):
    vec = lambda: pl.BlockSpec((1, D), lambda i: (0, 0))
    return pl.pallas_call(
        _c_update_body,
        grid=(NB,),
        in_specs=[
            pl.BlockSpec((BLK, D), lambda i: (i, 0)),
            pl.BlockSpec((BLK, SW), lambda i: (i, 0)),
            pl.BlockSpec((BLK, SW), lambda i: (NB + i, 0)),
            pl.BlockSpec((BLK, SW), lambda i: (i, 0)),
            pl.BlockSpec((8, D), lambda i: (0, 0)),
            pl.BlockSpec((2, NT, H), lambda i: (0, 0, 0)),
            vec(), vec(), vec(), vec(), vec(), vec(),
        ],
        out_specs=pl.BlockSpec((BLK, D), lambda i: (i, 0)),
        out_shape=jax.ShapeDtypeStruct((NP, D), F32),
    )(h, s_arr, s_arr, cnt, sums, m2, gm.reshape(1, D), bm.reshape(1, D),
      gu.reshape(1, D), bu.reshape(1, D), go.reshape(1, D), bo.reshape(1, D))


def _c_tables_body(h_r, pos_r, wnd_r, wns_r, wpd_r, wps_r, bd_r, td_r, ts_r):
    h = h_r[...]
    p = pos_r[...]
    td_r[...] = (jnp.dot(h, wnd_r[0], preferred_element_type=F32)
                 + jnp.dot(p, wpd_r[0], preferred_element_type=F32) + bd_r[0])
    ts_r[...] = (jnp.dot(h, wns_r[0], preferred_element_type=F32)
                 + jnp.dot(p, wps_r[0], preferred_element_type=F32))


def _c_tables(h, pos_pad, wnd, wns, wpd, wps, bd):
    return pl.pallas_call(
        _c_tables_body,
        grid=(2, NB),
        in_specs=[
            pl.BlockSpec((BLK, D), lambda c, i: (i, 0)),
            pl.BlockSpec((BLK, 4), lambda c, i: (i, 0)),
            pl.BlockSpec((1, D, D), lambda c, i: (c, 0, 0)),
            pl.BlockSpec((1, D, D), lambda c, i: (c, 0, 0)),
            pl.BlockSpec((1, 4, D), lambda c, i: (c, 0, 0)),
            pl.BlockSpec((1, 4, D), lambda c, i: (c, 0, 0)),
            pl.BlockSpec((1, 1, D), lambda c, i: (c, 0, 0)),
        ],
        out_specs=[
            pl.BlockSpec((BLK, D), lambda c, i: (c * NB + i, 0)),
            pl.BlockSpec((BLK, D), lambda c, i: (c * NB + i, 0)),
        ],
        out_shape=[
            jax.ShapeDtypeStruct((2 * NP, D), F32),
            jax.ShapeDtypeStruct((2 * NP, D), F32),
        ],
    )(h, pos_pad, wnd, wns, wpd, wps, bd.reshape(2, 1, D))


def _c_final_body(h_r, w1t_r, b1_r, w2t_r, b2_r, out_r, acc_r):
    i = pl.program_id(0)
    cs = jnp.sum(h_r[...], axis=0, keepdims=True)

    @pl.when(i == 0)
    def _():
        acc_r[...] = cs

    @pl.when(i > 0)
    def _():
        acc_r[...] = acc_r[...] + cs

    @pl.when(i == NB - 1)
    def _():
        hp = acc_r[...] / float(N)
        pre = jnp.dot(hp, w1t_r[...], preferred_element_type=F32) + b1_r[...]
        hid = jnp.maximum(pre, 0.0) + jnp.log1p(jnp.exp(-jnp.abs(pre)))
        out_r[...] = jnp.dot(hid, w2t_r[...], preferred_element_type=F32) + b2_r[...]


def _c_final(h, w1t, b1, w2t, b2):
    return pl.pallas_call(
        _c_final_body,
        grid=(NB,),
        in_specs=[
            pl.BlockSpec((BLK, D), lambda i: (i, 0)),
            pl.BlockSpec((D, 2 * D), lambda i: (0, 0)),
            pl.BlockSpec((1, 2 * D), lambda i: (0, 0)),
            pl.BlockSpec((2 * D, 1), lambda i: (0, 0)),
            pl.BlockSpec((1, 1), lambda i: (0, 0)),
        ],
        out_specs=pl.BlockSpec((1, 1), lambda i: (0, 0)),
        out_shape=jax.ShapeDtypeStruct((1, 1), F32),
        scratch_shapes=[pltpu.VMEM((1, D), F32)],
    )(h, w1t, b1, w2t, b2)


def kernel(pos, atomic_numbers, edge_index, emb, lin_f_W, lin_f_b, lin_s_W,
           lin_s_b, bn_msg_g, bn_msg_b, bn_upd_g, bn_upd_b, bn_out_g,
           bn_out_b, mlp_W1, mlp_b1, mlp_W2, mlp_b2):
    src = edge_index[0].astype(jnp.int32)
    dst = edge_index[1].astype(jnp.int32)
    ep = jnp.concatenate([dst.reshape(-1, CH), src.reshape(-1, CH)], axis=1)
    a_pad = jnp.pad(atomic_numbers.astype(jnp.int32), (0, NP - N))
    pos_pad = jnp.pad(pos.astype(F32), ((0, NP - N), (0, 1)))
    zer = jnp.zeros((NP, SW), F32)

    wnd, wns, wpd, wps, bdst = [], [], [], [], []
    for i in range(L):
        Wf, Ws = lin_f_W[i], lin_s_W[i]
        Wfd, Wfs, Wfe = Wf[:, :D].T, Wf[:, D:2 * D].T, Wf[:, 2 * D:].T
        Wsd, Wss, Wse = Ws[:, :D].T, Ws[:, D:2 * D].T, Ws[:, 2 * D:].T
        halves = lambda Wa, Wb: jnp.stack([
            jnp.concatenate([Wa[:, :H], Wb[:, :H]], axis=1),
            jnp.concatenate([Wa[:, H:], Wb[:, H:]], axis=1)])
        wnd.append(halves(Wfd, Wsd))
        wns.append(halves(Wfs, Wss))
        pad3 = lambda M: jnp.concatenate([M, jnp.zeros((1, D), F32)], axis=0)
        wpd.append(jnp.stack([
            pad3(jnp.concatenate([-Wfe[:, :H], -Wse[:, :H]], axis=1)),
            pad3(jnp.concatenate([-Wfe[:, H:], -Wse[:, H:]], axis=1))]))
        wps.append(-wpd[i])
        bf, bs = lin_f_b[i], lin_s_b[i]
        bdst.append(jnp.stack([
            jnp.concatenate([bf[:H], bs[:H]]),
            jnp.concatenate([bf[H:], bs[H:]])]))

    temb_d, temb_s = _c_emb(emb.astype(F32), wnd[0], wns[0], bdst[0])
    upd, ups = _c_pos(pos_pad, wpd[0], wps[0])
    h, tdst, tsrc = _sc_init(emb.astype(F32), temb_d, temb_s, a_pad, upd, ups)
    (cnt,) = _sc_count(dst, zer)

    for i in range(L):
        s_arr, m2 = _sc_edge(tdst, tsrc, ep, zer)
        m2 = m2.reshape(2, NT, H)
        sums = _c_stats(h, s_arr, cnt)
        h = _c_update(h, s_arr, cnt, sums, m2, bn_msg_g[i], bn_msg_b[i],
                      bn_upd_g[i], bn_upd_b[i], bn_out_g[i], bn_out_b[i])
        if i < L - 1:
            tdst, tsrc = _c_tables(h, pos_pad, wnd[i + 1], wns[i + 1],
                                   wpd[i + 1], wps[i + 1], bdst[i + 1])

    out = _c_final(h, mlp_W1.T.astype(F32), mlp_b1.reshape(1, 2 * D),
                   mlp_W2.T.astype(F32), mlp_b2.reshape(1, 1))
    return out.reshape(1)


# speedup vs baseline: 4.8286x; 3.9007x over previous
import jax
import jax.numpy as jnp
from jax import lax
from jax.experimental import pallas as pl
from jax.experimental.pallas import tpu as pltpu
from jax.experimental.pallas import tpu_sc as plsc

N = 10000
E = 160000
D = 256
H = 128
L = 3
NP = 10240
BLK = 512
NB = NP // BLK
SW = 128
NT = 16
NPT = NP // NT
EPT = E // NT
CH = 40
NCH = EPT // CH
EPS = 1e-5
F32 = jnp.float32

def _mesh():
    return plsc.VectorSubcoreMesh(core_axis_name="c", subcore_axis_name="s",
                                  num_cores=2, num_subcores=NT)


def _sc_init_body(temb_h, temb_d, temb_s, apad, upd, ups,
                  h0, t0d, t0s,
                  aidx, aadj, rows, urows, sem):
    c = lax.axis_index("c")
    s = lax.axis_index("s")
    nb = s * NPT

    def _add_rows(r, _):
        for g in range(D // 16):
            sl = pl.ds(g * 16, 16)
            rows[r, sl] = rows[r, sl] + urows[r, sl]
        return 0

    def chunk(j, _):
        off = nb + j * 128
        pltpu.sync_copy(apad.at[pl.ds(off, 128)], aidx)

        @pl.when(c == 0)
        def _():
            pltpu.async_copy(temb_h.at[aidx], rows, sem).wait()
            pltpu.sync_copy(rows, h0.at[pl.ds(off, 128)])

        for g in range(8):
            sl = pl.ds(g * 16, 16)
            aadj[sl] = aidx[sl] + c * 120

        pltpu.async_copy(temb_d.at[aadj], rows, sem).wait()
        pltpu.sync_copy(upd.at[pl.ds(c * NP + off, 128)], urows)
        lax.fori_loop(0, 128, _add_rows, 0)
        pltpu.sync_copy(rows, t0d.at[pl.ds(c * NP + off, 128)])

        pltpu.async_copy(temb_s.at[aadj], rows, sem).wait()
        pltpu.sync_copy(ups.at[pl.ds(c * NP + off, 128)], urows)
        lax.fori_loop(0, 128, _add_rows, 0)
        pltpu.sync_copy(rows, t0s.at[pl.ds(c * NP + off, 128)])
        return 0

    lax.fori_loop(0, NPT // 128, chunk, 0)


def _sc_edge_body(tdst, tsrc, ep, zer,
                  s_out, m2_out,
                  stab, pk0, pk1, dr0, dr1, da0, da1, sa0, sa1,
                  gd0, gd1, gs0, gs1, mb, acc, sem):
    c = lax.axis_index("c")
    s = lax.axis_index("s")

    pltpu.sync_copy(zer.at[pl.ds(s * NPT, NPT)], stab.at[pl.ds(s * NPT, NPT)])
    zv = jnp.zeros((16,), F32)
    for g in range(8):
        acc[pl.ds(g * 16, 16)] = zv
    plsc.subcore_barrier()

    cbase = s * NCH
    coff = c * NP
    P7 = (0.01000928961813237, -0.05243753706782591, 0.1308334279841901,
          -0.22316586411920608, 0.3272257149735533, -0.4992850491225031,
          0.9999670809438583, 2.55467301950837e-07)

    def load_idx(pk, dr, da, sa, gchunk):
        pltpu.sync_copy(ep.at[pl.ds(gchunk, 1)], pk)
        for o in (0, 16, 24):
            sl = pl.ds(o, 16)
            v = pk[0, sl]
            dr[sl] = v
            da[sl] = v + coff
            sa[sl] = pk[0, pl.ds(CH + o, 16)] + coff

    def issue(da, sa, gd, gs):
        pltpu.async_copy(tdst.at[da], gd, sem)
        pltpu.async_copy(tsrc.at[sa], gs, sem)

    def wait(da, sa, gd, gs):
        pltpu.make_async_copy(tdst.at[da], gd, sem).wait()
        pltpu.make_async_copy(tsrc.at[sa], gs, sem).wait()

    zero16 = jnp.zeros((16,), F32)

    def compute_scatter(gd, gs, dr):
        @plsc.parallel_loop(0, CH, unroll=1, carry=(zero16,) * 8)
        def a8(e, a8):
            G = range(8)
            fs = [gd[e, pl.ds(g * 16, 16)] + gs[e, pl.ds(g * 16, 16)] for g in G]
            svs = [gd[e, pl.ds(H + g * 16, 16)] + gs[e, pl.ds(H + g * 16, 16)]
                   for g in G]
            ef = [jnp.exp(-f) for f in fs]
            et = [jnp.exp(-jnp.abs(sv)) for sv in svs]
            inv = [1.0 / (1.0 + x) for x in ef]
            l1p = []
            for t in et:
                acc_p = jnp.full((16,), P7[0], F32)
                for ck in P7[1:]:
                    acc_p = acc_p * t + ck
                l1p.append(acc_p)
            out = []
            for g in G:
                m = (jnp.maximum(svs[g], 0.0) + l1p[g]) * inv[g]
                mb[e, pl.ds(g * 16, 16)] = m
                out.append(a8[g] + m * m)
            return tuple(out)

        for g in range(8):
            fo = pl.ds(g * 16, 16)
            acc[fo] = acc[fo] + a8[g]
        pltpu.sync_copy(mb, stab.at[dr], add=True)

    load_idx(pk0, dr0, da0, sa0, cbase)
    issue(da0, sa0, gd0, gs0)
    load_idx(pk1, dr1, da1, sa1, cbase + 1)

    def pair(u, _):
        a = cbase + 2 * u
        wait(da0, sa0, gd0, gs0)
        issue(da1, sa1, gd1, gs1)
        compute_scatter(gd0, gs0, dr0)

        @pl.when(2 * u + 2 < NCH)
        def _():
            load_idx(pk0, dr0, da0, sa0, a + 2)

        wait(da1, sa1, gd1, gs1)

        @pl.when(2 * u + 2 < NCH)
        def _():
            issue(da0, sa0, gd0, gs0)

        compute_scatter(gd1, gs1, dr1)

        @pl.when(2 * u + 3 < NCH)
        def _():
            load_idx(pk1, dr1, da1, sa1, a + 3)

        return 0

    lax.fori_loop(0, NCH // 2, pair, 0)
    plsc.subcore_barrier()

    pltpu.sync_copy(stab.at[pl.ds(s * NPT, NPT)],
                    s_out.at[pl.ds(coff + s * NPT, NPT)])
    pltpu.sync_copy(acc, m2_out.at[pl.ds((c * NT + s) * H, H)])


def _sc_count_body(dsti, zer, cnt_out, ctab, idr, ones, sem):
    c = lax.axis_index("c")
    s = lax.axis_index("s")

    @pl.when(c == 0)
    def _():
        pltpu.sync_copy(zer.at[pl.ds(s * NPT, NPT)],
                        ctab.at[pl.ds(s * NPT, NPT)])
        onev = jnp.where(lax.iota(jnp.int32, 16) == 0, 1.0, 0.0).astype(F32)
        zv = jnp.zeros((16,), F32)

        def _initrow(e, _):
            ones[e, pl.ds(0, 16)] = onev
            for g in range(1, 8):
                ones[e, pl.ds(g * 16, 16)] = zv
            return 0

        lax.fori_loop(0, CH, _initrow, 0)
        plsc.subcore_barrier()
        base = s * EPT

        def chunk(j, _):
            pltpu.sync_copy(dsti.at[pl.ds(base + j * CH, CH)], idr)
            pltpu.sync_copy(ones, ctab.at[idr], add=True)
            return 0

        lax.fori_loop(0, NCH, chunk, 0)
        plsc.subcore_barrier()
        pltpu.sync_copy(ctab.at[pl.ds(s * NPT, NPT)],
                        cnt_out.at[pl.ds(s * NPT, NPT)])


def _sc_count(dsti, zer):
    return pl.kernel(
        _sc_count_body,
        mesh=_mesh(),
        out_type=[jax.ShapeDtypeStruct((NP, SW), F32)],
        scratch_types=[
            pltpu.VMEM_SHARED((NP, SW), F32),
            pltpu.VMEM((CH,), jnp.int32),
            pltpu.VMEM((CH, SW), F32),
            pltpu.SemaphoreType.DMA,
        ],
    )(dsti, zer)


def _sc_init(temb_h, temb_d, temb_s, apad, upd, ups):
    return pl.kernel(
        _sc_init_body,
        mesh=_mesh(),
        out_type=[
            jax.ShapeDtypeStruct((NP, D), F32),
            jax.ShapeDtypeStruct((2 * NP, D), F32),
            jax.ShapeDtypeStruct((2 * NP, D), F32),
        ],
        scratch_types=[
            pltpu.VMEM((128,), jnp.int32),
            pltpu.VMEM((128,), jnp.int32),
            pltpu.VMEM((128, D), F32),
            pltpu.VMEM((128, D), F32),
            pltpu.SemaphoreType.DMA,
        ],
    )(temb_h, temb_d, temb_s, apad, upd, ups)


def _sc_edge(tdst, tsrc, ep, zer):
    idx32 = lambda: pltpu.VMEM((CH,), jnp.int32)
    gbuf = lambda: pltpu.VMEM((CH, D), F32)
    return pl.kernel(
        _sc_edge_body,
        mesh=_mesh(),
        out_type=[
            jax.ShapeDtypeStruct((2 * NP, SW), F32),
            jax.ShapeDtypeStruct((2 * NT * H,), F32),
        ],
        scratch_types=[
            pltpu.VMEM_SHARED((NP, SW), F32),
            pltpu.VMEM((1, 2 * CH), jnp.int32),
            pltpu.VMEM((1, 2 * CH), jnp.int32),
            idx32(), idx32(), idx32(), idx32(), idx32(), idx32(),
            gbuf(), gbuf(), gbuf(), gbuf(),
            pltpu.VMEM((CH, SW), F32),
            pltpu.VMEM((H,), F32),
            pltpu.SemaphoreType.DMA,
        ],
    )(tdst, tsrc, ep, zer)


def _mask_rows(i, x):
    rows = i * BLK + lax.broadcasted_iota(jnp.int32, (BLK, 1), 0)
    return jnp.where(rows < N, x, 0.0)


def _c_emb_body(emb_r, wnd_r, wns_r, bd_r, td_r, ts_r):
    e = emb_r[...]
    td_r[...] = jnp.dot(e, wnd_r[0], preferred_element_type=F32) + bd_r[0]
    ts_r[...] = jnp.dot(e, wns_r[0], preferred_element_type=F32)


def _c_emb(emb, wnd, wns, bd):
    return pl.pallas_call(
        _c_emb_body,
        grid=(2,),
        in_specs=[
            pl.BlockSpec((120, D), lambda c: (0, 0)),
            pl.BlockSpec((1, D, D), lambda c: (c, 0, 0)),
            pl.BlockSpec((1, D, D), lambda c: (c, 0, 0)),
            pl.BlockSpec((1, 1, D), lambda c: (c, 0, 0)),
        ],
        out_specs=[
            pl.BlockSpec((120, D), lambda c: (c, 0)),
            pl.BlockSpec((120, D), lambda c: (c, 0)),
        ],
        out_shape=[
            jax.ShapeDtypeStruct((240, D), F32),
            jax.ShapeDtypeStruct((240, D), F32),
        ],
    )(jnp.pad(emb, ((0, 2), (0, 0))), wnd, wns, bd.reshape(2, 1, D))


def _c_pos_body(pos_r, wpd_r, wps_r, ud_r, us_r):
    p = pos_r[...]
    ud_r[...] = jnp.dot(p, wpd_r[0], preferred_element_type=F32)
    us_r[...] = jnp.dot(p, wps_r[0], preferred_element_type=F32)


def _c_pos(pos_pad, wpd, wps):
    return pl.pallas_call(
        _c_pos_body,
        grid=(2, NB),
        in_specs=[
            pl.BlockSpec((BLK, 4), lambda c, i: (i, 0)),
            pl.BlockSpec((1, 4, D), lambda c, i: (c, 0, 0)),
            pl.BlockSpec((1, 4, D), lambda c, i: (c, 0, 0)),
        ],
        out_specs=[
            pl.BlockSpec((BLK, D), lambda c, i: (c * NB + i, 0)),
            pl.BlockSpec((BLK, D), lambda c, i: (c * NB + i, 0)),
        ],
        out_shape=[
            jax.ShapeDtypeStruct((2 * NP, D), F32),
            jax.ShapeDtypeStruct((2 * NP, D), F32),
        ],
    )(pos_pad, wpd, wps)


def _c_stats_body(h_r, s0_r, s1_r, cnt_r, out_r):
    i = pl.program_id(0)
    h = _mask_rows(i, h_r[...])
    S = jnp.concatenate([s0_r[...], s1_r[...]], axis=1)
    cnt = cnt_r[:, :1]
    st = jnp.stack([
        jnp.sum(h, axis=0),
        jnp.sum(h * h, axis=0),
        jnp.sum(h * S, axis=0),
        jnp.sum(h * cnt, axis=0),
        jnp.sum(S, axis=0),
        jnp.sum(S * S, axis=0),
        jnp.sum(S * cnt, axis=0),
        jnp.zeros((D,), F32) + jnp.sum(cnt * cnt),
    ])

    @pl.when(i == 0)
    def _():
        out_r[...] = st

    @pl.when(i > 0)
    def _():
        out_r[...] = out_r[...] + st


def _c_stats(h, s_arr, cnt):
    return pl.pallas_call(
        _c_stats_body,
        grid=(NB,),
        in_specs=[
            pl.BlockSpec((BLK, D), lambda i: (i, 0)),
            pl.BlockSpec((BLK, SW), lambda i: (i, 0)),
            pl.BlockSpec((BLK, SW), lambda i: (NB + i, 0)),
            pl.BlockSpec((BLK, SW), lambda i: (i, 0)),
        ],
        out_specs=pl.BlockSpec((8, D), lambda i: (0, 0)),
        out_shape=jax.ShapeDtypeStruct((8, D), F32),
    )(h, s_arr, s_arr, cnt)


def _coeffs(sums, m2, gm, bm, gu, bu, go, bo):
    Sh, Sh2, ShS, Shc, SS, SS2, SSc = (sums[k] for k in range(7))
    Cq = sums[7, 0]
    Ef = float(E)
    Nf = float(N)
    M2 = m2
    mu_m = SS / Ef
    vm = M2 / Ef - mu_m * mu_m
    k1 = gm / jnp.sqrt(vm + EPS)
    k2 = bm - mu_m * k1
    mu_a = (k1 * SS + k2 * Ef) / Nf
    Ea2 = (k1 * k1 * SS2 + 2.0 * k1 * k2 * SSc + k2 * k2 * Cq) / Nf
    va = Ea2 - mu_a * mu_a
    p1 = gu / jnp.sqrt(va + EPS)
    p2 = bu - mu_a * p1
    q1 = k1 * p1
    q2 = k2 * p1
    q3 = p2
    mu_u = (Sh + q1 * SS + q2 * Ef + Nf * q3) / Nf
    Eu2 = (Sh2 + q1 * q1 * SS2 + q2 * q2 * Cq + Nf * q3 * q3
           + 2.0 * q1 * ShS + 2.0 * q2 * Shc + 2.0 * q3 * Sh
           + 2.0 * q1 * q2 * SSc + 2.0 * q1 * q3 * SS + 2.0 * q2 * q3 * Ef) / Nf
    vu = Eu2 - mu_u * mu_u
    r1 = go / jnp.sqrt(vu + EPS)
    r2 = bo - mu_u * r1
    A = r1
    B = q1 * r1
    C = q2 * r1
    Dc = q3 * r1 + r2
    return A, B, C, Dc


def _c_update_body(h_r, s0_r, s1_r, cnt_r, sums_r, m2_r, gm_r, bm_r, gu_r,
                   bu_r, go_r, bo_r, out_r):
    i = pl.program_id(0)
    m2p = jnp.sum(m2_r[...], axis=1)
    m2 = jnp.concatenate([m2p[0], m2p[1]], axis=0)
    A, B, C, Dc = _coeffs(sums_r[...], m2, gm_r[0], bm_r[0], gu_r[0],
                          bu_r[0], go_r[0], bo_r[0])
    h = _mask_rows(i, h_r[...])
    S = jnp.concatenate([s0_r[...], s1_r[...]], axis=1)
    cnt = cnt_r[:, :1]
    u = h * A + S * B + cnt * C + Dc
    hn = jnp.maximum(u, 0.0) + jnp.log1p(jnp.exp(-jnp.abs(u)))
    out_r[...] = _mask_rows(i, hn)


def _c_update(h, s_arr, cnt, sums, m2, gm, bm, gu, bu, go, bo):
    vec = lambda: pl.BlockSpec((1, D), lambda i: (0, 0))
    return pl.pallas_call(
        _c_update_body,
        grid=(NB,),
        in_specs=[
            pl.BlockSpec((BLK, D), lambda i: (i, 0)),
            pl.BlockSpec((BLK, SW), lambda i: (i, 0)),
            pl.BlockSpec((BLK, SW), lambda i: (NB + i, 0)),
            pl.BlockSpec((BLK, SW), lambda i: (i, 0)),
            pl.BlockSpec((8, D), lambda i: (0, 0)),
            pl.BlockSpec((2, NT, H), lambda i: (0, 0, 0)),
            vec(), vec(), vec(), vec(), vec(), vec(),
        ],
        out_specs=pl.BlockSpec((BLK, D), lambda i: (i, 0)),
        out_shape=jax.ShapeDtypeStruct((NP, D), F32),
    )(h, s_arr, s_arr, cnt, sums, m2, gm.reshape(1, D), bm.reshape(1, D),
      gu.reshape(1, D), bu.reshape(1, D), go.reshape(1, D), bo.reshape(1, D))


def _c_tables_body(h_r, pos_r, wnd_r, wns_r, wpd_r, wps_r, bd_r, td_r, ts_r):
    h = h_r[...]
    p = pos_r[...]
    td_r[...] = (jnp.dot(h, wnd_r[0], preferred_element_type=F32)
                 + jnp.dot(p, wpd_r[0], preferred_element_type=F32) + bd_r[0])
    ts_r[...] = (jnp.dot(h, wns_r[0], preferred_element_type=F32)
                 + jnp.dot(p, wps_r[0], preferred_element_type=F32))


def _c_tables(h, pos_pad, wnd, wns, wpd, wps, bd):
    return pl.pallas_call(
        _c_tables_body,
        grid=(2, NB),
        in_specs=[
            pl.BlockSpec((BLK, D), lambda c, i: (i, 0)),
            pl.BlockSpec((BLK, 4), lambda c, i: (i, 0)),
            pl.BlockSpec((1, D, D), lambda c, i: (c, 0, 0)),
            pl.BlockSpec((1, D, D), lambda c, i: (c, 0, 0)),
            pl.BlockSpec((1, 4, D), lambda c, i: (c, 0, 0)),
            pl.BlockSpec((1, 4, D), lambda c, i: (c, 0, 0)),
            pl.BlockSpec((1, 1, D), lambda c, i: (c, 0, 0)),
        ],
        out_specs=[
            pl.BlockSpec((BLK, D), lambda c, i: (c * NB + i, 0)),
            pl.BlockSpec((BLK, D), lambda c, i: (c * NB + i, 0)),
        ],
        out_shape=[
            jax.ShapeDtypeStruct((2 * NP, D), F32),
            jax.ShapeDtypeStruct((2 * NP, D), F32),
        ],
    )(h, pos_pad, wnd, wns, wpd, wps, bd.reshape(2, 1, D))


def _c_final_body(h_r, w1t_r, b1_r, w2t_r, b2_r, out_r, acc_r):
    i = pl.program_id(0)
    cs = jnp.sum(h_r[...], axis=0, keepdims=True)

    @pl.when(i == 0)
    def _():
        acc_r[...] = cs

    @pl.when(i > 0)
    def _():
        acc_r[...] = acc_r[...] + cs

    @pl.when(i == NB - 1)
    def _():
        hp = acc_r[...] / float(N)
        pre = jnp.dot(hp, w1t_r[...], preferred_element_type=F32) + b1_r[...]
        hid = jnp.maximum(pre, 0.0) + jnp.log1p(jnp.exp(-jnp.abs(pre)))
        out_r[...] = jnp.dot(hid, w2t_r[...], preferred_element_type=F32) + b2_r[...]


def _c_final(h, w1t, b1, w2t, b2):
    return pl.pallas_call(
        _c_final_body,
        grid=(NB,),
        in_specs=[
            pl.BlockSpec((BLK, D), lambda i: (i, 0)),
            pl.BlockSpec((D, 2 * D), lambda i: (0, 0)),
            pl.BlockSpec((1, 2 * D), lambda i: (0, 0)),
            pl.BlockSpec((2 * D, 1), lambda i: (0, 0)),
            pl.BlockSpec((1, 1), lambda i: (0, 0)),
        ],
        out_specs=pl.BlockSpec((1, 1), lambda i: (0, 0)),
        out_shape=jax.ShapeDtypeStruct((1, 1), F32),
        scratch_shapes=[pltpu.VMEM((1, D), F32)],
    )(h, w1t, b1, w2t, b2)


def kernel(pos, atomic_numbers, edge_index, emb, lin_f_W, lin_f_b, lin_s_W,
           lin_s_b, bn_msg_g, bn_msg_b, bn_upd_g, bn_upd_b, bn_out_g,
           bn_out_b, mlp_W1, mlp_b1, mlp_W2, mlp_b2):
    src = edge_index[0].astype(jnp.int32)
    dst = edge_index[1].astype(jnp.int32)
    ep = jnp.concatenate([dst.reshape(-1, CH), src.reshape(-1, CH)], axis=1)
    a_pad = jnp.pad(atomic_numbers.astype(jnp.int32), (0, NP - N))
    pos_pad = jnp.pad(pos.astype(F32), ((0, NP - N), (0, 1)))
    zer = jnp.zeros((NP, SW), F32)

    wnd, wns, wpd, wps, bdst = [], [], [], [], []
    for i in range(L):
        Wf, Ws = lin_f_W[i], lin_s_W[i]
        Wfd, Wfs, Wfe = Wf[:, :D].T, Wf[:, D:2 * D].T, Wf[:, 2 * D:].T
        Wsd, Wss, Wse = Ws[:, :D].T, Ws[:, D:2 * D].T, Ws[:, 2 * D:].T
        halves = lambda Wa, Wb: jnp.stack([
            jnp.concatenate([Wa[:, :H], Wb[:, :H]], axis=1),
            jnp.concatenate([Wa[:, H:], Wb[:, H:]], axis=1)])
        wnd.append(halves(Wfd, Wsd))
        wns.append(halves(Wfs, Wss))
        pad3 = lambda M: jnp.concatenate([M, jnp.zeros((1, D), F32)], axis=0)
        wpd.append(jnp.stack([
            pad3(jnp.concatenate([-Wfe[:, :H], -Wse[:, :H]], axis=1)),
            pad3(jnp.concatenate([-Wfe[:, H:], -Wse[:, H:]], axis=1))]))
        wps.append(-wpd[i])
        bf, bs = lin_f_b[i], lin_s_b[i]
        bdst.append(jnp.stack([
            jnp.concatenate([bf[:H], bs[:H]]),
            jnp.concatenate([bf[H:], bs[H:]])]))

    temb_d, temb_s = _c_emb(emb.astype(F32), wnd[0], wns[0], bdst[0])
    upd, ups = _c_pos(pos_pad, wpd[0], wps[0])
    h, tdst, tsrc = _sc_init(emb.astype(F32), temb_d, temb_s, a_pad, upd, ups)
    (cnt,) = _sc_count(dst, zer)

    for i in range(L):
        s_arr, m2 = _sc_edge(tdst, tsrc, ep, zer)
        m2 = m2.reshape(2, NT, H)
        sums = _c_stats(h, s_arr, cnt)
        h = _c_update(h, s_arr, cnt, sums, m2, bn_msg_g[i], bn_msg_b[i],
                      bn_upd_g[i], bn_upd_b[i], bn_out_g[i], bn_out_b[i])
        if i < L - 1:
            tdst, tsrc = _c_tables(h, pos_pad, wnd[i + 1], wns[i + 1],
                                   wpd[i + 1], wps[i + 1], bdst[i + 1])

    out = _c_final(h, mlp_W1.T.astype(F32), mlp_b1.reshape(1, 2 * D),
                   mlp_W2.T.astype(F32), mlp_b2.reshape(1, 1))
    return out.reshape(1)
